# Initial kernel scaffold; baseline (speedup 1.0000x reference)
import jax
import jax.numpy as jnp
from jax.experimental import pallas as pl


def kernel(rel_embedding, T):
    raise NotImplementedError("write your pallas kernel here")


# speedup vs baseline: 1274.1589x; 1274.1589x over previous
import functools

import jax
import jax.numpy as jnp
from jax import lax
from jax.experimental import pallas as pl
from jax.experimental.pallas import tpu as pltpu
from jax.experimental.pallas import tpu_sc as plsc

MAXL2 = 4096
TBL = MAXL2 + 1
TBL_PAD = 4112
REV_ROW = 8208
N_PHASE = 8
ROWS_PER_W = 128
GROUPS = 16


def _sc_body(table_hbm, out_hbm, table_v, revp_v, sem):
    nc = 2
    wid = lax.axis_index("s") * nc + lax.axis_index("c")

    pltpu.sync_copy(table_hbm, table_v)

    iota = lax.iota(jnp.int32, 16)

    for p in range(N_PHASE):
        def build_body(c, carry, p=p):
            k0 = c * 16
            idx = jnp.clip((6143 - p - k0) - iota, 0, MAXL2)
            revp_v[pl.ds(p * REV_ROW + k0, 16)] = plsc.load_gather(table_v, [idx])
            return carry
        lax.fori_loop(0, REV_ROW // 16, build_body, 0)

    base = wid * ROWS_PER_W

    def group_body(g, carry):
        copies = []
        for b in range(8):
            i = base + g * 8 + b
            o = 4095 - i
            p = jnp.bitwise_and(o, 7)
            start = o - p
            flat = pl.multiple_of(p * REV_ROW + start, 8)
            copies.append(
                pltpu.async_copy(
                    revp_v.at[pl.ds(flat, MAXL2)],
                    out_hbm.at[pl.ds(i * MAXL2, MAXL2)],
                    sem,
                )
            )
        for cp in copies:
            cp.wait()
        return carry

    lax.fori_loop(0, GROUPS, group_body, 0)


@functools.partial(jax.jit, static_argnames=())
def _bias_sc(table_pad):
    mesh = plsc.VectorSubcoreMesh(core_axis_name="c", subcore_axis_name="s")
    out_flat = pl.kernel(
        _sc_body,
        out_type=jax.ShapeDtypeStruct((MAXL2 * MAXL2,), jnp.float32),
        mesh=mesh,
        compiler_params=pltpu.CompilerParams(needs_layout_passes=False),
        scratch_types=[
            pltpu.VMEM((TBL_PAD,), jnp.float32),
            pltpu.VMEM((N_PHASE * REV_ROW,), jnp.float32),
            pltpu.SemaphoreType.DMA,
        ],
    )(table_pad)
    return out_flat.reshape(MAXL2, MAXL2)


def kernel(rel_embedding, T):
    del T
    table_pad = jnp.concatenate(
        [rel_embedding, jnp.zeros((TBL_PAD - TBL,), jnp.float32)]
    )
    return _bias_sc(table_pad)


# speedup vs baseline: 3199.1945x; 2.5108x over previous
import functools

import jax
import jax.numpy as jnp
from jax import lax
from jax.experimental import pallas as pl
from jax.experimental.pallas import tpu as pltpu
from jax.experimental.pallas import tpu_sc as plsc

MAXL2 = 4096
TBL = MAXL2 + 1
TBL_PAD = 4112
N_CLASS = 4
REV_T = 64


def _sc_body(table_hbm, out_hbm, table_v, rev3d, sem):
    nc = 2
    wid = lax.axis_index("s") * nc + lax.axis_index("c")

    pltpu.sync_copy(table_hbm, table_v)

    iota = lax.iota(jnp.int32, 16)

    for q in range(N_CLASS):
        cls = wid * N_CLASS + q
        r = lax.shift_right_logical(cls, 4)
        m = jnp.bitwise_and(cls, 15)
        phi = jnp.bitwise_and(4095 - r - 8 * m, 127)

        def build_body(t, carry, q=q, phi=phi):
            s0 = (6143 - 128 * t) - phi
            for cc in range(8):
                idx = jnp.clip((s0 - 16 * cc) - iota, 0, MAXL2)
                rev3d[q, t, pl.ds(16 * cc, 16)] = plsc.load_gather(table_v, [idx])
            return carry

        lax.fori_loop(0, REV_T, build_body, 0)

        def group_body(g, carry, q=q, r=r, m=m):
            copies = []
            for u in range(8):
                s = g * 8 + u
                b = 16 * s + m
                t0 = 31 - s
                copies.append(
                    pltpu.async_copy(
                        rev3d.at[q, pl.ds(t0, 32), :],
                        out_hbm.at[b, :, r, :],
                        sem,
                    )
                )
            for cp in copies:
                cp.wait()
            return carry

        lax.fori_loop(0, 4, group_body, 0)


@jax.jit
def _bias_sc(table_pad):
    mesh = plsc.VectorSubcoreMesh(core_axis_name="c", subcore_axis_name="s")
    out4 = pl.kernel(
        _sc_body,
        out_type=jax.ShapeDtypeStruct((512, 32, 8, 128), jnp.float32),
        mesh=mesh,
        compiler_params=pltpu.CompilerParams(needs_layout_passes=False),
        scratch_types=[
            pltpu.VMEM((TBL_PAD,), jnp.float32),
            pltpu.VMEM((N_CLASS, REV_T, 128), jnp.float32),
            pltpu.SemaphoreType.DMA,
        ],
    )(table_pad)
    return out4.transpose(0, 2, 1, 3).reshape(MAXL2, MAXL2)


def kernel(rel_embedding, T):
    del T
    table_pad = jnp.concatenate(
        [rel_embedding, jnp.zeros((TBL_PAD - TBL,), jnp.float32)]
    )
    return _bias_sc(table_pad)


# speedup vs baseline: 3602.5211x; 1.1261x over previous
import jax
import jax.numpy as jnp
from jax import lax
from jax.experimental import pallas as pl
from jax.experimental.pallas import tpu as pltpu
from jax.experimental.pallas import tpu_sc as plsc

MAXL2 = 4096
TBL = MAXL2 + 1
N_CLASS = 4
REV_T = 64


def _sc_body(table_hbm, out_hbm, table_v, rev3d, sem):
    nc = 2
    wid = lax.axis_index("s") * nc + lax.axis_index("c")

    pltpu.sync_copy(table_hbm, table_v)

    iota = lax.iota(jnp.int32, 16)

    def drain_class(_g, carry):
        for _u in range(8):
            pltpu.make_async_copy(
                out_hbm.at[0, :, 0, :], rev3d.at[0, pl.ds(0, 32), :], sem
            ).wait()
        return carry

    for q in range(N_CLASS):
        cls = wid * N_CLASS + q
        r = lax.shift_right_logical(cls, 4)
        m = jnp.bitwise_and(cls, 15)
        phi = jnp.bitwise_and(4095 - r - 8 * m, 127)

        def build_body(t, carry, q=q, phi=phi):
            s0 = (6143 - 128 * t) - phi
            for cc in range(8):
                idx = jnp.clip((s0 - 16 * cc) - iota, 0, MAXL2)
                rev3d[q, t, pl.ds(16 * cc, 16)] = plsc.load_gather(table_v, [idx])
            return carry

        lax.fori_loop(0, REV_T, build_body, 0)

        if q > 0:
            lax.fori_loop(0, 4, drain_class, 0)

        def fire_body(g, carry, q=q, r=r, m=m):
            for u in range(8):
                s = g * 8 + u
                b = 16 * s + m
                t0 = 31 - s
                pltpu.async_copy(
                    rev3d.at[q, pl.ds(t0, 32), :],
                    out_hbm.at[b, :, r, :],
                    sem,
                )
            return carry

        lax.fori_loop(0, 4, fire_body, 0)

    lax.fori_loop(0, 4, drain_class, 0)


@jax.jit
def _bias_sc(table):
    mesh = plsc.VectorSubcoreMesh(core_axis_name="c", subcore_axis_name="s")
    out4 = pl.kernel(
        _sc_body,
        out_type=jax.ShapeDtypeStruct((512, 32, 8, 128), jnp.float32),
        mesh=mesh,
        compiler_params=pltpu.CompilerParams(needs_layout_passes=False),
        scratch_types=[
            pltpu.VMEM((TBL,), jnp.float32),
            pltpu.VMEM((N_CLASS, REV_T, 128), jnp.float32),
            pltpu.SemaphoreType.DMA,
        ],
    )(table)
    return out4.transpose(0, 2, 1, 3).reshape(MAXL2, MAXL2)


def kernel(rel_embedding, T):
    del T
    return _bias_sc(rel_embedding)
